# Initial kernel scaffold; baseline (speedup 1.0000x reference)
import jax
import jax.numpy as jnp
from jax.experimental import pallas as pl


def kernel(input_ids, modality_type, table, pos_emb, mod_emb, gamma, beta):
    raise NotImplementedError("write your pallas kernel here")


# speedup vs baseline: 1.2114x; 1.2114x over previous
import functools

import jax
import jax.numpy as jnp
from jax import lax
from jax.experimental import pallas as pl
from jax.experimental.pallas import tpu as pltpu
from jax.experimental.pallas import tpu_sc as plsc

B = 1024
L = 512
D = 768
N_TOK = B * L
EPS = 1e-5

_NC = 2
_NS = 16
_NW = _NC * _NS
_RPW = N_TOK // _NW
_CHUNK = 128
_NCHUNK = _RPW // _CHUNK


def _sc_gather_body(ids_hbm, table_hbm, out_hbm, idx_v, rows_v, sem):
    wid = lax.axis_index("s") * _NC + lax.axis_index("c")
    base = wid * _RPW

    def body(i, carry):
        off = base + i * _CHUNK
        pltpu.sync_copy(ids_hbm.at[pl.ds(off, _CHUNK)], idx_v)
        pltpu.async_copy(table_hbm.at[idx_v], rows_v, sem).wait()
        pltpu.sync_copy(rows_v, out_hbm.at[pl.ds(off, _CHUNK)])
        return carry

    lax.fori_loop(0, _NCHUNK, body, 0)


_sc_gather = functools.partial(
    pl.kernel,
    mesh=plsc.VectorSubcoreMesh(core_axis_name="c", subcore_axis_name="s"),
    out_type=jax.ShapeDtypeStruct((N_TOK, D), jnp.float32),
    scratch_types=[
        pltpu.VMEM((_CHUNK,), jnp.int32),
        pltpu.VMEM((_CHUNK, D), jnp.float32),
        pltpu.SemaphoreType.DMA,
    ],
)(_sc_gather_body)


def _ln_body(x_ref, add_ref, gamma_ref, beta_ref, o_ref):
    x = x_ref[...] + add_ref[...]
    m = jnp.mean(x, axis=-1, keepdims=True)
    d = x - m
    v = jnp.mean(d * d, axis=-1, keepdims=True)
    o_ref[...] = d * lax.rsqrt(v + EPS) * gamma_ref[...] + beta_ref[...]


def _ln_call(buf, addvec, gamma2d, beta2d):
    return pl.pallas_call(
        _ln_body,
        grid=(N_TOK // L,),
        in_specs=[
            pl.BlockSpec((L, D), lambda i: (i, 0)),
            pl.BlockSpec((L, D), lambda i: (0, 0)),
            pl.BlockSpec((1, D), lambda i: (0, 0)),
            pl.BlockSpec((1, D), lambda i: (0, 0)),
        ],
        out_specs=pl.BlockSpec((L, D), lambda i: (i, 0)),
        out_shape=jax.ShapeDtypeStruct((N_TOK, D), jnp.float32),
    )(buf, addvec, gamma2d, beta2d)


def kernel(input_ids, modality_type, table, pos_emb, mod_emb, gamma, beta):
    ids = input_ids.reshape(N_TOK).astype(jnp.int32)
    buf = _sc_gather(ids, table)
    mod_row = lax.dynamic_index_in_dim(mod_emb, modality_type, axis=0,
                                       keepdims=False)
    addvec = pos_emb[0, :L, :] + mod_row[None, :]
    out = _ln_call(buf, addvec, gamma.reshape(1, D), beta.reshape(1, D))
    return out.reshape(B, L, D)


# speedup vs baseline: 1.3456x; 1.1108x over previous
import functools

import jax
import jax.numpy as jnp
from jax import lax
from jax.experimental import pallas as pl
from jax.experimental.pallas import tpu as pltpu
from jax.experimental.pallas import tpu_sc as plsc

B = 1024
L = 512
D = 768
N_TOK = B * L
EPS = 1e-5

_NC = 2
_NS = 16
_NW = _NC * _NS
_SEG = 4
_SEG_ROWS = N_TOK // _SEG
_RPW = _SEG_ROWS // _NW
_CHUNK = 128
_NCHUNK = _RPW // _CHUNK


def _sc_gather_body(ids_hbm, table_hbm, out_hbm, idx_v, rows_v, sem):
    wid = lax.axis_index("s") * _NC + lax.axis_index("c")
    base = wid * _RPW

    def body(i, carry):
        off = base + i * _CHUNK
        pltpu.sync_copy(ids_hbm.at[pl.ds(off, _CHUNK)], idx_v)
        pltpu.async_copy(table_hbm.at[idx_v], rows_v, sem).wait()
        pltpu.sync_copy(rows_v, out_hbm.at[pl.ds(off, _CHUNK)])
        return carry

    lax.fori_loop(0, _NCHUNK, body, 0)


_sc_gather = functools.partial(
    pl.kernel,
    mesh=plsc.VectorSubcoreMesh(core_axis_name="c", subcore_axis_name="s"),
    out_type=jax.ShapeDtypeStruct((_SEG_ROWS, D), jnp.float32),
    scratch_types=[
        pltpu.VMEM((_CHUNK,), jnp.int32),
        pltpu.VMEM((_CHUNK, D), jnp.float32),
        pltpu.SemaphoreType.DMA,
    ],
)(_sc_gather_body)


def _ln_first_body(x_ref, add_ref, gamma_ref, beta_ref, o_ref):
    x = x_ref[...] + add_ref[...]
    m = jnp.mean(x, axis=-1, keepdims=True)
    d = x - m
    v = jnp.mean(d * d, axis=-1, keepdims=True)
    o_ref[...] = d * lax.rsqrt(v + EPS) * gamma_ref[...] + beta_ref[...]


def _ln_body(x_ref, add_ref, gamma_ref, beta_ref, acc_ref, o_ref):
    del acc_ref
    _ln_first_body(x_ref, add_ref, gamma_ref, beta_ref, o_ref)


def _ln_seg(seg, buf, addvec, gamma2d, beta2d, acc):
    base_blk = seg * (_SEG_ROWS // L)
    return pl.pallas_call(
        _ln_body,
        grid=(_SEG_ROWS // L,),
        in_specs=[
            pl.BlockSpec((L, D), lambda i: (i, 0)),
            pl.BlockSpec((L, D), lambda i: (0, 0)),
            pl.BlockSpec((1, D), lambda i: (0, 0)),
            pl.BlockSpec((1, D), lambda i: (0, 0)),
            pl.BlockSpec(memory_space=pl.ANY),
        ],
        out_specs=pl.BlockSpec((L, D), lambda i: (base_blk + i, 0)),
        out_shape=jax.ShapeDtypeStruct((N_TOK, D), jnp.float32),
        input_output_aliases={4: 0},
    )(buf, addvec, gamma2d, beta2d, acc)


def _ln_first(buf, addvec, gamma2d, beta2d):
    return pl.pallas_call(
        _ln_first_body,
        grid=(_SEG_ROWS // L,),
        in_specs=[
            pl.BlockSpec((L, D), lambda i: (i, 0)),
            pl.BlockSpec((L, D), lambda i: (0, 0)),
            pl.BlockSpec((1, D), lambda i: (0, 0)),
            pl.BlockSpec((1, D), lambda i: (0, 0)),
        ],
        out_specs=pl.BlockSpec((L, D), lambda i: (i, 0)),
        out_shape=jax.ShapeDtypeStruct((N_TOK, D), jnp.float32),
    )(buf, addvec, gamma2d, beta2d)


def kernel(input_ids, modality_type, table, pos_emb, mod_emb, gamma, beta):
    ids = input_ids.reshape(N_TOK).astype(jnp.int32)
    mod_row = lax.dynamic_index_in_dim(mod_emb, modality_type, axis=0,
                                       keepdims=False)
    addvec = pos_emb[0, :L, :] + mod_row[None, :]
    gamma2d = gamma.reshape(1, D)
    beta2d = beta.reshape(1, D)

    bufs = [_sc_gather(lax.dynamic_slice_in_dim(ids, s * _SEG_ROWS,
                                                _SEG_ROWS), table)
            for s in range(_SEG)]
    acc = _ln_first(bufs[0], addvec, gamma2d, beta2d)
    for s in range(1, _SEG):
        acc = _ln_seg(s, bufs[s], addvec, gamma2d, beta2d, acc)
    return acc.reshape(B, L, D)


# speedup vs baseline: 1.3692x; 1.0175x over previous
import functools

import jax
import jax.numpy as jnp
from jax import lax
from jax.experimental import pallas as pl
from jax.experimental.pallas import tpu as pltpu
from jax.experimental.pallas import tpu_sc as plsc

B = 1024
L = 512
D = 768
N_TOK = B * L
EPS = 1e-5

_NC = 2
_NS = 16
_NW = _NC * _NS
_SEG = 8
_SEG_ROWS = N_TOK // _SEG
_RPW = _SEG_ROWS // _NW
_CHUNK = 64
_NCHUNK = _RPW // _CHUNK


def _sc_gather_body(ids_hbm, table_hbm, out_hbm, idx_v, rows0, rows1,
                    gsem0, gsem1):
    wid = lax.axis_index("s") * _NC + lax.axis_index("c")
    base = wid * _RPW
    cbase = wid * _NCHUNK

    def out_at(i):
        return out_hbm.at[pl.ds(base + i * _CHUNK, _CHUNK)]

    pltpu.sync_copy(ids_hbm.at[pl.ds(cbase, _NCHUNK)], idx_v)

    def gather(i_buf, idx_row, sem):
        return pltpu.async_copy(table_hbm.at[idx_v.at[idx_row]], i_buf, sem)

    gather(rows0, 0, gsem0)

    def pair(j, carry):
        i0 = 2 * j
        pltpu.make_async_copy(table_hbm.at[idx_v.at[i0]], rows0, gsem0).wait()
        gather(rows1, i0 + 1, gsem1)
        pltpu.sync_copy(rows0, out_at(i0))
        pltpu.make_async_copy(table_hbm.at[idx_v.at[i0 + 1]], rows1,
                              gsem1).wait()
        gather(rows0, i0 + 2, gsem0)
        pltpu.sync_copy(rows1, out_at(i0 + 1))
        return carry

    lax.fori_loop(0, _NCHUNK // 2 - 1, pair, 0)

    i0 = _NCHUNK - 2
    pltpu.make_async_copy(table_hbm.at[idx_v.at[i0]], rows0, gsem0).wait()
    gather(rows1, i0 + 1, gsem1)
    pltpu.sync_copy(rows0, out_at(i0))
    pltpu.make_async_copy(table_hbm.at[idx_v.at[i0 + 1]], rows1, gsem1).wait()
    pltpu.sync_copy(rows1, out_at(i0 + 1))


_sc_gather = functools.partial(
    pl.kernel,
    mesh=plsc.VectorSubcoreMesh(core_axis_name="c", subcore_axis_name="s"),
    out_type=jax.ShapeDtypeStruct((_SEG_ROWS, D), jnp.float32),
    scratch_types=[
        pltpu.VMEM((_NCHUNK, _CHUNK), jnp.int32),
        pltpu.VMEM((_CHUNK, D), jnp.float32),
        pltpu.VMEM((_CHUNK, D), jnp.float32),
        pltpu.SemaphoreType.DMA,
        pltpu.SemaphoreType.DMA,
    ],
)(_sc_gather_body)


def _ln_first_body(x_ref, add_ref, gamma_ref, beta_ref, o_ref):
    x = x_ref[...] + add_ref[...]
    m = jnp.mean(x, axis=-1, keepdims=True)
    d = x - m
    v = jnp.mean(d * d, axis=-1, keepdims=True)
    o_ref[...] = d * lax.rsqrt(v + EPS) * gamma_ref[...] + beta_ref[...]


def _ln_body(x_ref, add_ref, gamma_ref, beta_ref, acc_ref, o_ref):
    del acc_ref
    _ln_first_body(x_ref, add_ref, gamma_ref, beta_ref, o_ref)


def _ln_seg(seg, buf, addvec, gamma2d, beta2d, acc):
    base_blk = seg * (_SEG_ROWS // L)
    return pl.pallas_call(
        _ln_body,
        grid=(_SEG_ROWS // L,),
        in_specs=[
            pl.BlockSpec((L, D), lambda i: (i, 0)),
            pl.BlockSpec((L, D), lambda i: (0, 0)),
            pl.BlockSpec((1, D), lambda i: (0, 0)),
            pl.BlockSpec((1, D), lambda i: (0, 0)),
            pl.BlockSpec(memory_space=pl.ANY),
        ],
        out_specs=pl.BlockSpec((L, D), lambda i: (base_blk + i, 0)),
        out_shape=jax.ShapeDtypeStruct((N_TOK, D), jnp.float32),
        input_output_aliases={4: 0},
    )(buf, addvec, gamma2d, beta2d, acc)


def _ln_first(buf, addvec, gamma2d, beta2d):
    return pl.pallas_call(
        _ln_first_body,
        grid=(_SEG_ROWS // L,),
        in_specs=[
            pl.BlockSpec((L, D), lambda i: (i, 0)),
            pl.BlockSpec((L, D), lambda i: (0, 0)),
            pl.BlockSpec((1, D), lambda i: (0, 0)),
            pl.BlockSpec((1, D), lambda i: (0, 0)),
        ],
        out_specs=pl.BlockSpec((L, D), lambda i: (i, 0)),
        out_shape=jax.ShapeDtypeStruct((N_TOK, D), jnp.float32),
    )(buf, addvec, gamma2d, beta2d)


def kernel(input_ids, modality_type, table, pos_emb, mod_emb, gamma, beta):
    ids = input_ids.reshape(N_TOK // _CHUNK, _CHUNK).astype(jnp.int32)
    mod_row = lax.dynamic_index_in_dim(mod_emb, modality_type, axis=0,
                                       keepdims=False)
    addvec = pos_emb[0, :L, :] + mod_row[None, :]
    gamma2d = gamma.reshape(1, D)
    beta2d = beta.reshape(1, D)

    seg_id_rows = _SEG_ROWS // _CHUNK
    bufs = [_sc_gather(lax.dynamic_slice_in_dim(ids, s * seg_id_rows,
                                                seg_id_rows), table)
            for s in range(_SEG)]
    acc = _ln_first(bufs[0], addvec, gamma2d, beta2d)
    for s in range(1, _SEG):
        acc = _ln_seg(s, bufs[s], addvec, gamma2d, beta2d, acc)
    return acc.reshape(B, L, D)
